# Initial kernel scaffold; baseline (speedup 1.0000x reference)
import jax
import jax.numpy as jnp
from jax.experimental import pallas as pl


def kernel(inc_node_edge, x_h, e_h, edge_in_W, edge_in_b, gnn_params, edge_out_W, edge_out_b):
    raise NotImplementedError("write your pallas kernel here")


# speedup vs baseline: 2.1793x; 2.1793x over previous
import functools

import jax
import jax.numpy as jnp
from jax import lax
from jax.experimental import pallas as pl
from jax.experimental.pallas import tpu as pltpu
from jax.experimental.pallas import tpu_sc as plsc

_N = 10000
_E = 320000
_C = 80
_f32 = jnp.float32
_NCH0 = _E // 16 // _C
_NW = 32
_NCH = _E // _NW // _C

_L0_ROWS = 5120
_L0_ACC = _L0_ROWS + _C
_PK_ROWS = 2560


def _sc_layer0():
    mesh = plsc.VectorSubcoreMesh(core_axis_name="c", subcore_axis_name="s")

    @functools.partial(
        pl.kernel,
        mesh=mesh,
        out_type=jax.ShapeDtypeStruct((2, _L0_ROWS, 128), _f32),
        scratch_types=[
            pltpu.VMEM((_NCH0, _C), jnp.int32),
            pltpu.VMEM((_NCH0, _C), jnp.int32),
            pltpu.VMEM((_C, 128), _f32),
            pltpu.VMEM((_C, 128), _f32),
            pltpu.VMEM_SHARED((_L0_ACC, 128), _f32),
            pltpu.SemaphoreType.DMA,
        ],
    )
    def k(x_hbm, el_hbm, src_hbm, dst_hbm, out_hbm,
          src_v, dst_v, xrows, erows, acc, sem):
        cid = lax.axis_index("c")
        sid = lax.axis_index("s")
        pltpu.sync_copy(src_hbm.at[sid], src_v)
        pltpu.sync_copy(dst_hbm.at[sid], dst_v)

        lo = cid * 5000
        iota = lax.iota(jnp.int32, 16)

        def fix(i, _):
            for g in range(_C // 16):
                sl = pl.ds(g * 16, 16)
                d = dst_v[i, sl] - lo
                ok = jnp.logical_and(d >= 0, d < 5000)
                dst_v[i, sl] = jnp.where(ok, d, _L0_ROWS + g * 16 + iota)
            return 0
        lax.fori_loop(0, _NCH0, fix, 0)

        def zer(j, _):
            for g in range(8):
                erows[j, pl.ds(g * 16, 16)] = jnp.zeros((16,), _f32)
            return 0
        lax.fori_loop(0, _C, zer, 0)
        base = sid * (_L0_ROWS // 16)
        for kk in range(_L0_ROWS // 16 // _C):
            pltpu.sync_copy(erows, acc.at[pl.ds(base + kk * _C, _C)])
        plsc.subcore_barrier()

        def chunk(i, _):
            pltpu.async_copy(x_hbm.at[src_v.at[i]], xrows, sem).wait()
            pltpu.sync_copy(el_hbm.at[sid, i], erows)

            def edge(j, _):
                for g in range(8):
                    sl = pl.ds(g * 16, 16)
                    erows[j, sl] = jnp.maximum(xrows[j, sl] + erows[j, sl],
                                               0.0)
                return 0
            lax.fori_loop(0, _C, edge, 0)
            pltpu.sync_copy(erows, acc.at[dst_v.at[i]], add=True)
            return 0
        lax.fori_loop(0, _NCH0, chunk, 0)
        plsc.subcore_barrier()
        pltpu.sync_copy(acc.at[pl.ds(base, _L0_ROWS // 16)],
                        out_hbm.at[cid, pl.ds(base, _L0_ROWS // 16)])

    return k


def _sc_layer12():
    mesh = plsc.VectorSubcoreMesh(core_axis_name="c", subcore_axis_name="s")

    @functools.partial(
        pl.kernel,
        mesh=mesh,
        out_type=jax.ShapeDtypeStruct((2, _PK_ROWS, 128), _f32),
        scratch_types=[
            pltpu.VMEM((_NCH, _C), jnp.int32),
            pltpu.VMEM((_NCH, _C), jnp.int32),
            pltpu.VMEM((_C // 8, 128), jnp.int32),
            pltpu.VMEM((_C, 128), _f32),
            pltpu.VMEM((_C // 4, 128), _f32),
            pltpu.VMEM((_C, 128), _f32),
            pltpu.VMEM_SHARED((_PK_ROWS, 128), _f32),
            pltpu.SemaphoreType.DMA,
        ],
    )
    def k(xrep_hbm, el_hbm, src_hbm, dstq_hbm, dstm_hbm, out_hbm,
          src_v, dstq_v, dstm_v, xrows, elb, erows, acc, sem):
        cid = lax.axis_index("c")
        sid = lax.axis_index("s")
        w = sid * 2 + cid
        pltpu.sync_copy(src_hbm.at[w], src_v)
        pltpu.sync_copy(dstq_hbm.at[w], dstq_v)

        def zer(j, _):
            for g in range(8):
                erows[j, pl.ds(g * 16, 16)] = jnp.zeros((16,), _f32)
            return 0
        lax.fori_loop(0, _C, zer, 0)
        base = sid * (_PK_ROWS // 16)
        for kk in range(_PK_ROWS // 16 // _C):
            pltpu.sync_copy(erows, acc.at[pl.ds(base + kk * _C, _C)])
        plsc.subcore_barrier()

        zero = jnp.zeros((16,), _f32)

        def chunk(i, _):
            pltpu.async_copy(xrep_hbm.at[src_v.at[i]], xrows, sem).wait()
            pltpu.sync_copy(el_hbm.at[w, i], elb)
            pltpu.sync_copy(dstm_hbm.at[w, i], dstm_v)

            def edge(j, _):
                dmg = dstm_v[j // 8, pl.ds((j % 8) * 16, 16)]
                erow = j // 4
                ecol = (j % 4) * 32
                m0 = jnp.maximum(
                    xrows[j, pl.ds(0, 16)] + elb[erow, pl.ds(ecol, 16)], 0.0)
                m1 = jnp.maximum(
                    xrows[j, pl.ds(16, 16)]
                    + elb[erow, pl.ds(ecol + 16, 16)], 0.0)
                for wd in range(4):
                    msk = dmg == wd * 32
                    erows[j, pl.ds(wd * 32, 16)] = jnp.where(msk, m0, zero)
                    erows[j, pl.ds(wd * 32 + 16, 16)] = jnp.where(msk, m1,
                                                                  zero)
                return 0
            lax.fori_loop(0, _C, edge, 0)
            pltpu.sync_copy(erows, acc.at[dstq_v.at[i]], add=True)
            return 0
        lax.fori_loop(0, _NCH, chunk, 0)
        plsc.subcore_barrier()
        pltpu.sync_copy(acc.at[pl.ds(base, _PK_ROWS // 16)],
                        out_hbm.at[cid, pl.ds(base, _PK_ROWS // 16)])

    return k


def _full(shape):
    return pl.BlockSpec(shape, lambda i: tuple(0 for _ in shape))


def _fold_body(einW, einb, we0, be0, we1, be1, we2, be2, eoW, eob,
               wc0, bc0, wc1, bc1, wc2, bc2, wo, bo):
    W = einW[...]
    b = einb[...]
    for we, be, wc, bc in ((we0, be0, wc0, bc0), (we1, be1, wc1, bc1),
                           (we2, be2, wc2, bc2), (eoW, eob, wo, bo)):
        m = we[...]
        wc[...] = jnp.dot(W, m, preferred_element_type=_f32)
        bc[...] = jnp.dot(b, m, preferred_element_type=_f32) + be[...]


def _fold_weights(einW, einb, gnn_params, eoW, eob):
    ins = [einW, einb.reshape(1, -1)]
    outs = []
    specs = [_full((16, 128)), _full((1, 128))]
    for p in gnn_params:
        d = p["We"].shape[1]
        ins += [p["We"], p["be"].reshape(1, -1)]
        specs += [_full((128, d)), _full((1, d))]
        outs += [jax.ShapeDtypeStruct((16, d), _f32),
                 jax.ShapeDtypeStruct((1, d), _f32)]
    ins += [eoW, eob.reshape(1, -1)]
    specs += [_full((128, 16)), _full((1, 16))]
    outs += [jax.ShapeDtypeStruct((16, 16), _f32),
             jax.ShapeDtypeStruct((1, 16), _f32)]
    out_specs = [_full(o.shape) for o in outs]
    return pl.pallas_call(
        _fold_body, grid=(1,), in_specs=specs, out_specs=out_specs,
        out_shape=outs)(*ins)


_BE = 3200


def _edge_body(eh_ref, wc0, bc0, wc1, bc1, wc2, bc2, wo, bo,
               el0, el1, el2, eo):
    eh = eh_ref[...]
    el0[...] = jnp.dot(eh, wc0[...], preferred_element_type=_f32) + bc0[...]
    el1[...] = jnp.dot(eh, wc1[...], preferred_element_type=_f32) + bc1[...]
    el2[...] = jnp.dot(eh, wc2[...], preferred_element_type=_f32) + bc2[...]
    v = jnp.dot(eh, wo[...], preferred_element_type=_f32) + bo[...]
    up = jnp.concatenate([v[-1:], v[:-1]], axis=0)
    down = jnp.concatenate([v[1:], v[:1]], axis=0)
    row = lax.broadcasted_iota(jnp.int32, v.shape, 0)
    eo[...] = 0.5 * (v + jnp.where(row % 2 == 1, up, down))


def _edge_features(e_h, wc0, bc0, wc1, bc1, wc2, bc2, wo, bo):
    nblk = _E // _BE
    espec = pl.BlockSpec((_BE, 16), lambda i: (i, 0))
    wspecs = [_full(w.shape) for w in (wc0, bc0, wc1, bc1, wc2, bc2, wo, bo)]
    outs = [jax.ShapeDtypeStruct((_E, 128), _f32),
            jax.ShapeDtypeStruct((_E, 32), _f32),
            jax.ShapeDtypeStruct((_E, 32), _f32),
            jax.ShapeDtypeStruct((_E, 16), _f32)]
    out_specs = [pl.BlockSpec((_BE, o.shape[1]), lambda i: (i, 0))
                 for o in outs]
    return pl.pallas_call(
        _edge_body, grid=(nblk,), in_specs=[espec] + wspecs,
        out_specs=out_specs, out_shape=outs)(
            e_h, wc0, bc0, wc1, bc1, wc2, bc2, wo, bo)


_BN = 2000


def _node_body(x_ref, a0_ref, a1_ref, w1, b1, w2, b2, out_ref, *, use_gelu,
               single):
    if single:
        t = x_ref[...] + a0_ref[...]
    else:
        t = x_ref[...] + a0_ref[...] + a1_ref[...]
    h = jnp.dot(t, w1[...], preferred_element_type=_f32) + b1[...]
    h = jnp.maximum(h, 0.0)
    h = jnp.dot(h, w2[...], preferred_element_type=_f32) + b2[...]
    if use_gelu:
        h = 0.5 * h * (1.0 + lax.erf(h * 0.7071067811865476))
    out_ref[...] = h


def _node_update(x, a0, a1, W1, b1, W2, b2, use_gelu):
    din = x.shape[1]
    dhid = W1.shape[1]
    dout = W2.shape[1]
    nblk = _N // _BN
    bspec = lambda d: pl.BlockSpec((_BN, d), lambda i: (i, 0))
    single = a1 is None
    args = [x, a0] + ([] if single else [a1])
    in_specs = [bspec(din), bspec(din)] + ([] if single else [bspec(din)])
    if single:
        def body(x_ref, a0_ref, w1, b1, w2, b2, out_ref):
            _node_body(x_ref, a0_ref, None, w1, b1, w2, b2, out_ref,
                       use_gelu=use_gelu, single=True)
    else:
        body = functools.partial(_node_body, use_gelu=use_gelu, single=False)
    return pl.pallas_call(
        body,
        grid=(nblk,),
        in_specs=in_specs + [_full((din, dhid)), _full((1, dhid)),
                             _full((dhid, dout)), _full((1, dout))],
        out_specs=bspec(dout),
        out_shape=jax.ShapeDtypeStruct((_N, dout), _f32))(
            *args, W1, b1.reshape(1, -1), W2, b2.reshape(1, -1))


_sc_l0 = _sc_layer0()
_sc_l12 = _sc_layer12()


def kernel(inc_node_edge, x_h, e_h, edge_in_W, edge_in_b, gnn_params,
           edge_out_W, edge_out_b):
    src = inc_node_edge[:, 0]
    dst = inc_node_edge[:, 1]
    src16 = src.reshape(16, _NCH0, _C)
    dst16 = dst.reshape(16, _NCH0, _C)
    src32 = src.reshape(_NW, _NCH, _C)
    dstq = (dst // 4).reshape(_NW, _NCH, _C)
    dstm = jnp.broadcast_to(((dst % 4) * 32)[:, None], (_E, 16))
    dstm = dstm.reshape(_NW, _NCH, _C // 8, 128)

    (wc0, bc0, wc1, bc1, wc2, bc2, wo, bo) = _fold_weights(
        edge_in_W, edge_in_b, gnn_params, edge_out_W, edge_out_b)

    el0, el1, el2, eo = _edge_features(e_h, wc0, bc0, wc1, bc1, wc2, bc2,
                                       wo, bo)

    p = gnn_params[0]
    agg = _sc_l0(x_h, el0.reshape(16, _NCH0, _C, 128), src16, dst16)
    agg = agg[:, :5000].reshape(_N, 128)
    x = _node_update(x_h, agg, None, p["W1"], p["b1"], p["W2"], p["b2"],
                     use_gelu=True)
    xs = [x]

    for l in (1, 2):
        p = gnn_params[l]
        el = (el1 if l == 1 else el2).reshape(_NW, _NCH, _C // 4, 128)
        xrep = jnp.tile(x, (1, 4))
        ag = _sc_l12(xrep, el, src32, dstq, dstm)
        a0 = ag[0].reshape(4 * _PK_ROWS, 32)[:_N]
        a1 = ag[1].reshape(4 * _PK_ROWS, 32)[:_N]
        x = _node_update(x, a0, a1, p["W1"], p["b1"], p["W2"], p["b2"],
                         use_gelu=(l < 2))
        xs.append(x)

    return (x, eo, tuple(xs))


# speedup vs baseline: 3.1628x; 1.4513x over previous
import functools

import jax
import jax.numpy as jnp
from jax import lax
from jax.experimental import pallas as pl
from jax.experimental.pallas import tpu as pltpu
from jax.experimental.pallas import tpu_sc as plsc

_N = 10000
_E = 320000
_C = 80
_f32 = jnp.float32
_NCH0 = _E // 16 // _C
_NW = 32
_NCH = _E // _NW // _C

_L0_ROWS = 5120
_L0_ACC = _L0_ROWS + _C
_PK_ROWS = 2560


def _sc_layer0():
    mesh = plsc.VectorSubcoreMesh(core_axis_name="c", subcore_axis_name="s")

    _HC = _NCH0 // 2

    @functools.partial(
        pl.kernel,
        mesh=mesh,
        out_type=jax.ShapeDtypeStruct((2, _L0_ROWS, 128), _f32),
        scratch_types=[
            pltpu.VMEM((_HC, _C), jnp.int32),
            pltpu.VMEM((_HC, _C), jnp.int32),
            pltpu.VMEM((_C, 128), _f32),
            pltpu.VMEM((_C, 128), _f32),
            pltpu.VMEM((_C, 128), _f32),
            pltpu.VMEM((_C, 128), _f32),
            pltpu.VMEM_SHARED((_L0_ACC, 128), _f32),
            pltpu.SemaphoreType.DMA,
            pltpu.SemaphoreType.DMA,
        ],
    )
    def k(x_hbm, el_hbm, src_hbm, dst_hbm, out_hbm,
          src_v, dst_v, xr0, xr1, er0, er1, acc, si0, si1):
        cid = lax.axis_index("c")
        sid = lax.axis_index("s")
        lo = cid * 5000
        iota = lax.iota(jnp.int32, 16)

        def zer(j, _):
            for g in range(8):
                er0[j, pl.ds(g * 16, 16)] = jnp.zeros((16,), _f32)
            return 0
        lax.fori_loop(0, _C, zer, 0)
        base = sid * (_L0_ROWS // 16)
        for kk in range(_L0_ROWS // 16 // _C):
            pltpu.sync_copy(er0, acc.at[pl.ds(base + kk * _C, _C)])
        plsc.subcore_barrier()

        def issue_in(h, i, xr, er, si):
            pltpu.async_copy(x_hbm.at[src_v.at[i]], xr, si)
            pltpu.async_copy(el_hbm.at[sid, h * _HC + i], er, si)

        def wait_in(h, i, xr, er, si):
            pltpu.make_async_copy(x_hbm.at[src_v.at[i]], xr, si).wait()
            pltpu.make_async_copy(el_hbm.at[sid, h * _HC + i], er, si).wait()

        def compute(xr, er):
            def edge(j, _):
                for g in range(8):
                    sl = pl.ds(g * 16, 16)
                    er[j, sl] = jnp.maximum(xr[j, sl] + er[j, sl], 0.0)
                return 0
            lax.fori_loop(0, _C, edge, 0)

        for h in (0, 1):
            pltpu.sync_copy(src_hbm.at[sid, h], src_v)
            pltpu.sync_copy(dst_hbm.at[sid, h], dst_v)

            def fix(i, _):
                for g in range(_C // 16):
                    sl = pl.ds(g * 16, 16)
                    d = dst_v[i, sl] - lo
                    ok = jnp.logical_and(d >= 0, d < 5000)
                    dst_v[i, sl] = jnp.where(ok, d,
                                             _L0_ROWS + g * 16 + iota)
                return 0
            lax.fori_loop(0, _HC, fix, 0)

            issue_in(h, 0, xr0, er0, si0)

            def pair(i2, _):
                c0 = 2 * i2
                issue_in(h, c0 + 1, xr1, er1, si1)
                wait_in(h, c0, xr0, er0, si0)
                compute(xr0, er0)
                pltpu.sync_copy(er0, acc.at[dst_v.at[c0]], add=True)

                @pl.when(i2 < _HC // 2 - 1)
                def _():
                    issue_in(h, c0 + 2, xr0, er0, si0)
                wait_in(h, c0 + 1, xr1, er1, si1)
                compute(xr1, er1)
                pltpu.sync_copy(er1, acc.at[dst_v.at[c0 + 1]], add=True)
                return 0
            lax.fori_loop(0, _HC // 2, pair, 0)
            ct = _HC - 1
            issue_in(h, ct, xr0, er0, si0)
            wait_in(h, ct, xr0, er0, si0)
            compute(xr0, er0)
            pltpu.sync_copy(er0, acc.at[dst_v.at[ct]], add=True)

        plsc.subcore_barrier()
        pltpu.sync_copy(acc.at[pl.ds(base, _L0_ROWS // 16)],
                        out_hbm.at[cid, pl.ds(base, _L0_ROWS // 16)])

    return k


def _sc_layer12():
    mesh = plsc.VectorSubcoreMesh(core_axis_name="c", subcore_axis_name="s")

    @functools.partial(
        pl.kernel,
        mesh=mesh,
        out_type=jax.ShapeDtypeStruct((2, _PK_ROWS, 128), _f32),
        scratch_types=[
            pltpu.VMEM((_NCH, _C), jnp.int32),
            pltpu.VMEM((_NCH, _C), jnp.int32),
            pltpu.VMEM((_C // 8, 128), jnp.int32),
            pltpu.VMEM((_C, 128), _f32),
            pltpu.VMEM((_C // 4, 128), _f32),
            pltpu.VMEM_SHARED((_PK_ROWS, 128), _f32),
            pltpu.SemaphoreType.DMA,
        ],
    )
    def k(xrep_hbm, el_hbm, src_hbm, dstq_hbm, dstm_hbm, out_hbm,
          src_v, dstq_v, dm0, xr0, eb0, acc, si0):
        cid = lax.axis_index("c")
        sid = lax.axis_index("s")
        w = sid * 2 + cid
        pltpu.sync_copy(src_hbm.at[w], src_v)
        pltpu.sync_copy(dstq_hbm.at[w], dstq_v)

        def zer(j, _):
            for g in range(8):
                xr0[j, pl.ds(g * 16, 16)] = jnp.zeros((16,), _f32)
            return 0
        lax.fori_loop(0, _C, zer, 0)
        base = sid * (_PK_ROWS // 16)
        for kk in range(_PK_ROWS // 16 // _C):
            pltpu.sync_copy(xr0, acc.at[pl.ds(base + kk * _C, _C)])
        plsc.subcore_barrier()

        zero = jnp.zeros((16,), _f32)

        def compute(xr, eb, dm):
            def edge(j, _):
                dmg = dm[j // 8, pl.ds((j % 8) * 16, 16)]
                erow = j // 4
                ecol = (j % 4) * 32
                m0 = jnp.maximum(
                    xr[j, pl.ds(0, 16)] + eb[erow, pl.ds(ecol, 16)], 0.0)
                m1 = jnp.maximum(
                    xr[j, pl.ds(16, 16)] + eb[erow, pl.ds(ecol + 16, 16)],
                    0.0)
                for wd in range(4):
                    msk = dmg == wd * 32
                    xr[j, pl.ds(wd * 32, 16)] = jnp.where(msk, m0, zero)
                    xr[j, pl.ds(wd * 32 + 16, 16)] = jnp.where(msk, m1, zero)
                return 0
            lax.fori_loop(0, _C, edge, 0)

        def chunk(i, _):
            h = pltpu.async_copy(xrep_hbm.at[src_v.at[i]], xr0, si0)
            pltpu.sync_copy(el_hbm.at[w, i], eb0)
            pltpu.sync_copy(dstm_hbm.at[w, i], dm0)
            h.wait()
            compute(xr0, eb0, dm0)
            pltpu.sync_copy(xr0, acc.at[dstq_v.at[i]], add=True)
            return 0
        lax.fori_loop(0, _NCH, chunk, 0)
        plsc.subcore_barrier()
        pltpu.sync_copy(acc.at[pl.ds(base, _PK_ROWS // 16)],
                        out_hbm.at[cid, pl.ds(base, _PK_ROWS // 16)])

    return k


def _full(shape):
    return pl.BlockSpec(shape, lambda i: tuple(0 for _ in shape))


def _fold_body(einW, einb, we0, be0, we1, be1, we2, be2, eoW, eob,
               wc0, bc0, wc1, bc1, wc2, bc2, wo, bo):
    W = einW[...]
    b = einb[...]
    for we, be, wc, bc in ((we0, be0, wc0, bc0), (we1, be1, wc1, bc1),
                           (we2, be2, wc2, bc2), (eoW, eob, wo, bo)):
        m = we[...]
        wc[...] = jnp.dot(W, m, preferred_element_type=_f32)
        bc[...] = jnp.dot(b, m, preferred_element_type=_f32) + be[...]


def _fold_weights(einW, einb, gnn_params, eoW, eob):
    ins = [einW, einb.reshape(1, -1)]
    outs = []
    specs = [_full((16, 128)), _full((1, 128))]
    for p in gnn_params:
        d = p["We"].shape[1]
        ins += [p["We"], p["be"].reshape(1, -1)]
        specs += [_full((128, d)), _full((1, d))]
        outs += [jax.ShapeDtypeStruct((16, d), _f32),
                 jax.ShapeDtypeStruct((1, d), _f32)]
    ins += [eoW, eob.reshape(1, -1)]
    specs += [_full((128, 16)), _full((1, 16))]
    outs += [jax.ShapeDtypeStruct((16, 16), _f32),
             jax.ShapeDtypeStruct((1, 16), _f32)]
    out_specs = [_full(o.shape) for o in outs]
    return pl.pallas_call(
        _fold_body, grid=(1,), in_specs=specs, out_specs=out_specs,
        out_shape=outs)(*ins)


_BE = 3200


def _edge_body(eh_ref, wc0, bc0, wc1, bc1, wc2, bc2, wo, bo,
               el0, el1, el2, eo):
    eh = eh_ref[...]
    el0[...] = jnp.dot(eh, wc0[...], preferred_element_type=_f32) + bc0[...]
    el1[...] = jnp.dot(eh, wc1[...], preferred_element_type=_f32) + bc1[...]
    el2[...] = jnp.dot(eh, wc2[...], preferred_element_type=_f32) + bc2[...]
    v = jnp.dot(eh, wo[...], preferred_element_type=_f32) + bo[...]
    up = jnp.concatenate([v[-1:], v[:-1]], axis=0)
    down = jnp.concatenate([v[1:], v[:1]], axis=0)
    row = lax.broadcasted_iota(jnp.int32, v.shape, 0)
    eo[...] = 0.5 * (v + jnp.where(row % 2 == 1, up, down))


def _edge_features(e_h, wc0, bc0, wc1, bc1, wc2, bc2, wo, bo):
    nblk = _E // _BE
    espec = pl.BlockSpec((_BE, 16), lambda i: (i, 0))
    wspecs = [_full(w.shape) for w in (wc0, bc0, wc1, bc1, wc2, bc2, wo, bo)]
    outs = [jax.ShapeDtypeStruct((_E, 128), _f32),
            jax.ShapeDtypeStruct((_E, 32), _f32),
            jax.ShapeDtypeStruct((_E, 32), _f32),
            jax.ShapeDtypeStruct((_E, 16), _f32)]
    out_specs = [pl.BlockSpec((_BE, o.shape[1]), lambda i: (i, 0))
                 for o in outs]
    return pl.pallas_call(
        _edge_body, grid=(nblk,), in_specs=[espec] + wspecs,
        out_specs=out_specs, out_shape=outs)(
            e_h, wc0, bc0, wc1, bc1, wc2, bc2, wo, bo)


_BN = 2000


def _node_body(x_ref, a0_ref, a1_ref, w1, b1, w2, b2, out_ref, *, use_gelu,
               single):
    if single:
        t = x_ref[...] + a0_ref[...]
    else:
        t = x_ref[...] + a0_ref[...] + a1_ref[...]
    h = jnp.dot(t, w1[...], preferred_element_type=_f32) + b1[...]
    h = jnp.maximum(h, 0.0)
    h = jnp.dot(h, w2[...], preferred_element_type=_f32) + b2[...]
    if use_gelu:
        h = 0.5 * h * (1.0 + lax.erf(h * 0.7071067811865476))
    out_ref[...] = h


def _node_update(x, a0, a1, W1, b1, W2, b2, use_gelu):
    din = x.shape[1]
    dhid = W1.shape[1]
    dout = W2.shape[1]
    nblk = _N // _BN
    bspec = lambda d: pl.BlockSpec((_BN, d), lambda i: (i, 0))
    single = a1 is None
    args = [x, a0] + ([] if single else [a1])
    in_specs = [bspec(din), bspec(din)] + ([] if single else [bspec(din)])
    if single:
        def body(x_ref, a0_ref, w1, b1, w2, b2, out_ref):
            _node_body(x_ref, a0_ref, None, w1, b1, w2, b2, out_ref,
                       use_gelu=use_gelu, single=True)
    else:
        body = functools.partial(_node_body, use_gelu=use_gelu, single=False)
    return pl.pallas_call(
        body,
        grid=(nblk,),
        in_specs=in_specs + [_full((din, dhid)), _full((1, dhid)),
                             _full((dhid, dout)), _full((1, dout))],
        out_specs=bspec(dout),
        out_shape=jax.ShapeDtypeStruct((_N, dout), _f32))(
            *args, W1, b1.reshape(1, -1), W2, b2.reshape(1, -1))


_sc_l0 = _sc_layer0()
_sc_l12 = _sc_layer12()


def kernel(inc_node_edge, x_h, e_h, edge_in_W, edge_in_b, gnn_params,
           edge_out_W, edge_out_b):
    src = inc_node_edge[:, 0]
    dst = inc_node_edge[:, 1]
    src16 = src.reshape(16, 2, _NCH0 // 2, _C)
    dst16 = dst.reshape(16, 2, _NCH0 // 2, _C)
    src32 = src.reshape(_NW, _NCH, _C)
    dstq = (dst // 4).reshape(_NW, _NCH, _C)
    dstm = jnp.broadcast_to(((dst % 4) * 32)[:, None], (_E, 16))
    dstm = dstm.reshape(_NW, _NCH, _C // 8, 128)

    (wc0, bc0, wc1, bc1, wc2, bc2, wo, bo) = _fold_weights(
        edge_in_W, edge_in_b, gnn_params, edge_out_W, edge_out_b)

    el0, el1, el2, eo = _edge_features(e_h, wc0, bc0, wc1, bc1, wc2, bc2,
                                       wo, bo)

    p = gnn_params[0]
    agg = _sc_l0(x_h, el0.reshape(16, _NCH0, _C, 128), src16, dst16)
    agg = agg[:, :5000].reshape(_N, 128)
    x = _node_update(x_h, agg, None, p["W1"], p["b1"], p["W2"], p["b2"],
                     use_gelu=True)
    xs = [x]

    for l in (1, 2):
        p = gnn_params[l]
        el = (el1 if l == 1 else el2).reshape(_NW, _NCH, _C // 4, 128)
        xrep = jnp.tile(x, (1, 4))
        ag = _sc_l12(xrep, el, src32, dstq, dstm)
        a0 = ag[0].reshape(4 * _PK_ROWS, 32)[:_N]
        a1 = ag[1].reshape(4 * _PK_ROWS, 32)[:_N]
        x = _node_update(x, a0, a1, p["W1"], p["b1"], p["W2"], p["b2"],
                         use_gelu=(l < 2))
        xs.append(x)

    return (x, eo, tuple(xs))


# speedup vs baseline: 4.0225x; 1.2718x over previous
import functools

import jax
import jax.numpy as jnp
from jax import lax
from jax.experimental import pallas as pl
from jax.experimental.pallas import tpu as pltpu
from jax.experimental.pallas import tpu_sc as plsc

_N = 10000
_E = 320000
_C = 80
_f32 = jnp.float32
_NCH0 = _E // 16 // _C
_NW = 32
_NCH = _E // _NW // _C

_L0_ROWS = 5120
_L0_ACC = _L0_ROWS + _C
_PK_ROWS = 2560


def _sc_layer0():
    mesh = plsc.VectorSubcoreMesh(core_axis_name="c", subcore_axis_name="s")

    _HC = _NCH0 // 2

    @functools.partial(
        pl.kernel,
        mesh=mesh,
        out_type=jax.ShapeDtypeStruct((2, _L0_ROWS, 128), _f32),
        scratch_types=[
            pltpu.VMEM((_HC, _C), jnp.int32),
            pltpu.VMEM((_HC, _C), jnp.int32),
            pltpu.VMEM((_C, 128), _f32),
            pltpu.VMEM((_C, 128), _f32),
            pltpu.VMEM((_C, 128), _f32),
            pltpu.VMEM((_C, 128), _f32),
            pltpu.VMEM_SHARED((_L0_ACC, 128), _f32),
            pltpu.SemaphoreType.DMA,
            pltpu.SemaphoreType.DMA,
        ],
    )
    def k(x_hbm, el_hbm, src_hbm, dst_hbm, out_hbm,
          src_v, dst_v, xr0, xr1, er0, er1, acc, si0, si1):
        cid = lax.axis_index("c")
        sid = lax.axis_index("s")
        lo = cid * 5000
        iota = lax.iota(jnp.int32, 16)

        def zer(j, _):
            for g in range(8):
                er0[j, pl.ds(g * 16, 16)] = jnp.zeros((16,), _f32)
            return 0
        lax.fori_loop(0, _C, zer, 0)
        base = sid * (_L0_ROWS // 16)
        for kk in range(_L0_ROWS // 16 // _C):
            pltpu.sync_copy(er0, acc.at[pl.ds(base + kk * _C, _C)])
        plsc.subcore_barrier()

        def issue_in(h, i, xr, er, si):
            pltpu.async_copy(x_hbm.at[src_v.at[i]], xr, si)
            pltpu.async_copy(el_hbm.at[sid, h * _HC + i], er, si)

        def wait_in(h, i, xr, er, si):
            pltpu.make_async_copy(x_hbm.at[src_v.at[i]], xr, si).wait()
            pltpu.make_async_copy(el_hbm.at[sid, h * _HC + i], er, si).wait()

        def compute(xr, er):
            def edge(j, _):
                for g in range(8):
                    sl = pl.ds(g * 16, 16)
                    er[j, sl] = jnp.maximum(xr[j, sl] + er[j, sl], 0.0)
                return 0
            lax.fori_loop(0, _C, edge, 0)

        for h in (0, 1):
            pltpu.sync_copy(src_hbm.at[sid, h], src_v)
            pltpu.sync_copy(dst_hbm.at[sid, h], dst_v)

            def fix(i, _):
                for g in range(_C // 16):
                    sl = pl.ds(g * 16, 16)
                    d = dst_v[i, sl] - lo
                    ok = jnp.logical_and(d >= 0, d < 5000)
                    dst_v[i, sl] = jnp.where(ok, d,
                                             _L0_ROWS + g * 16 + iota)
                return 0
            lax.fori_loop(0, _HC, fix, 0)

            issue_in(h, 0, xr0, er0, si0)

            def pair(i2, _):
                c0 = 2 * i2
                issue_in(h, c0 + 1, xr1, er1, si1)
                wait_in(h, c0, xr0, er0, si0)
                compute(xr0, er0)
                pltpu.sync_copy(er0, acc.at[dst_v.at[c0]], add=True)

                @pl.when(i2 < _HC // 2 - 1)
                def _():
                    issue_in(h, c0 + 2, xr0, er0, si0)
                wait_in(h, c0 + 1, xr1, er1, si1)
                compute(xr1, er1)
                pltpu.sync_copy(er1, acc.at[dst_v.at[c0 + 1]], add=True)
                return 0
            lax.fori_loop(0, _HC // 2, pair, 0)
            ct = _HC - 1
            issue_in(h, ct, xr0, er0, si0)
            wait_in(h, ct, xr0, er0, si0)
            compute(xr0, er0)
            pltpu.sync_copy(er0, acc.at[dst_v.at[ct]], add=True)

        plsc.subcore_barrier()
        pltpu.sync_copy(acc.at[pl.ds(base, _L0_ROWS // 16)],
                        out_hbm.at[cid, pl.ds(base, _L0_ROWS // 16)])

    return k


def _sc_layer12():
    mesh = plsc.VectorSubcoreMesh(core_axis_name="c", subcore_axis_name="s")

    @functools.partial(
        pl.kernel,
        mesh=mesh,
        out_type=jax.ShapeDtypeStruct((2, _PK_ROWS, 128), _f32),
        scratch_types=[
            pltpu.VMEM((_NCH, _C), jnp.int32),
            pltpu.VMEM((_NCH, _C), jnp.int32),
            pltpu.VMEM((_C // 8, 128), jnp.int32),
            pltpu.VMEM((_C // 8, 128), jnp.int32),
            pltpu.VMEM((_C, 128), _f32),
            pltpu.VMEM((_C, 128), _f32),
            pltpu.VMEM((_C // 4, 128), _f32),
            pltpu.VMEM((_C // 4, 128), _f32),
            pltpu.VMEM_SHARED((_PK_ROWS, 128), _f32),
            pltpu.SemaphoreType.DMA,
            pltpu.SemaphoreType.DMA,
        ],
    )
    def k(xrep_hbm, el_hbm, src_hbm, dstq_hbm, dstm_hbm, out_hbm,
          src_v, dstq_v, dm0, dm1, xr0, xr1, eb0, eb1, acc, si0, si1):
        cid = lax.axis_index("c")
        sid = lax.axis_index("s")
        w = sid * 2 + cid
        pltpu.sync_copy(src_hbm.at[w], src_v)
        pltpu.sync_copy(dstq_hbm.at[w], dstq_v)

        def zer(j, _):
            for g in range(8):
                xr0[j, pl.ds(g * 16, 16)] = jnp.zeros((16,), _f32)
            return 0
        lax.fori_loop(0, _C, zer, 0)
        base = sid * (_PK_ROWS // 16)
        for kk in range(_PK_ROWS // 16 // _C):
            pltpu.sync_copy(xr0, acc.at[pl.ds(base + kk * _C, _C)])
        plsc.subcore_barrier()

        zero = jnp.zeros((16,), _f32)

        def compute(xr, eb, dm):
            def edge(j, _):
                dmg = dm[j // 8, pl.ds((j % 8) * 16, 16)]
                erow = j // 4
                ecol = (j % 4) * 32
                m0 = jnp.maximum(
                    xr[j, pl.ds(0, 16)] + eb[erow, pl.ds(ecol, 16)], 0.0)
                m1 = jnp.maximum(
                    xr[j, pl.ds(16, 16)] + eb[erow, pl.ds(ecol + 16, 16)],
                    0.0)
                for wd in range(4):
                    msk = dmg == wd * 32
                    xr[j, pl.ds(wd * 32, 16)] = jnp.where(msk, m0, zero)
                    xr[j, pl.ds(wd * 32 + 16, 16)] = jnp.where(msk, m1, zero)
                return 0
            lax.fori_loop(0, _C, edge, 0)

        def issue_in(i, xr, eb, dm, si):
            pltpu.async_copy(xrep_hbm.at[src_v.at[i]], xr, si)
            pltpu.async_copy(el_hbm.at[w, i], eb, si)
            pltpu.async_copy(dstm_hbm.at[w, i], dm, si)

        def wait_in(i, xr, eb, dm, si):
            pltpu.make_async_copy(xrep_hbm.at[src_v.at[i]], xr, si).wait()
            pltpu.make_async_copy(el_hbm.at[w, i], eb, si).wait()
            pltpu.make_async_copy(dstm_hbm.at[w, i], dm, si).wait()

        issue_in(0, xr0, eb0, dm0, si0)

        def pair(i2, _):
            c0 = 2 * i2
            issue_in(c0 + 1, xr1, eb1, dm1, si1)
            wait_in(c0, xr0, eb0, dm0, si0)
            compute(xr0, eb0, dm0)
            pltpu.sync_copy(xr0, acc.at[dstq_v.at[c0]], add=True)

            @pl.when(i2 < _NCH // 2 - 1)
            def _():
                issue_in(c0 + 2, xr0, eb0, dm0, si0)
            wait_in(c0 + 1, xr1, eb1, dm1, si1)
            compute(xr1, eb1, dm1)
            pltpu.sync_copy(xr1, acc.at[dstq_v.at[c0 + 1]], add=True)
            return 0
        lax.fori_loop(0, _NCH // 2, pair, 0)
        ct = _NCH - 1
        issue_in(ct, xr0, eb0, dm0, si0)
        wait_in(ct, xr0, eb0, dm0, si0)
        compute(xr0, eb0, dm0)
        pltpu.sync_copy(xr0, acc.at[dstq_v.at[ct]], add=True)
        plsc.subcore_barrier()
        pltpu.sync_copy(acc.at[pl.ds(base, _PK_ROWS // 16)],
                        out_hbm.at[cid, pl.ds(base, _PK_ROWS // 16)])

    return k


def _full(shape):
    return pl.BlockSpec(shape, lambda i: tuple(0 for _ in shape))


def _fold_body(einW, einb, we0, be0, we1, be1, we2, be2, eoW, eob,
               wc0, bc0, wc1, bc1, wc2, bc2, wo, bo):
    W = einW[...]
    b = einb[...]
    for we, be, wc, bc in ((we0, be0, wc0, bc0), (we1, be1, wc1, bc1),
                           (we2, be2, wc2, bc2), (eoW, eob, wo, bo)):
        m = we[...]
        wc[...] = jnp.dot(W, m, preferred_element_type=_f32)
        bc[...] = jnp.dot(b, m, preferred_element_type=_f32) + be[...]


def _fold_weights(einW, einb, gnn_params, eoW, eob):
    ins = [einW, einb.reshape(1, -1)]
    outs = []
    specs = [_full((16, 128)), _full((1, 128))]
    for p in gnn_params:
        d = p["We"].shape[1]
        ins += [p["We"], p["be"].reshape(1, -1)]
        specs += [_full((128, d)), _full((1, d))]
        outs += [jax.ShapeDtypeStruct((16, d), _f32),
                 jax.ShapeDtypeStruct((1, d), _f32)]
    ins += [eoW, eob.reshape(1, -1)]
    specs += [_full((128, 16)), _full((1, 16))]
    outs += [jax.ShapeDtypeStruct((16, 16), _f32),
             jax.ShapeDtypeStruct((1, 16), _f32)]
    out_specs = [_full(o.shape) for o in outs]
    return pl.pallas_call(
        _fold_body, grid=(1,), in_specs=specs, out_specs=out_specs,
        out_shape=outs)(*ins)


_BE = 3200


def _edge_body(eh_ref, wc0, bc0, wc1, bc1, wc2, bc2, wo, bo,
               el0, el1, el2, eo):
    eh = eh_ref[...]
    el0[...] = jnp.dot(eh, wc0[...], preferred_element_type=_f32) + bc0[...]
    el1[...] = jnp.dot(eh, wc1[...], preferred_element_type=_f32) + bc1[...]
    el2[...] = jnp.dot(eh, wc2[...], preferred_element_type=_f32) + bc2[...]
    v = jnp.dot(eh, wo[...], preferred_element_type=_f32) + bo[...]
    up = jnp.concatenate([v[-1:], v[:-1]], axis=0)
    down = jnp.concatenate([v[1:], v[:1]], axis=0)
    row = lax.broadcasted_iota(jnp.int32, v.shape, 0)
    eo[...] = 0.5 * (v + jnp.where(row % 2 == 1, up, down))


def _edge_features(e_h, wc0, bc0, wc1, bc1, wc2, bc2, wo, bo):
    nblk = _E // _BE
    espec = pl.BlockSpec((_BE, 16), lambda i: (i, 0))
    wspecs = [_full(w.shape) for w in (wc0, bc0, wc1, bc1, wc2, bc2, wo, bo)]
    outs = [jax.ShapeDtypeStruct((_E, 128), _f32),
            jax.ShapeDtypeStruct((_E, 32), _f32),
            jax.ShapeDtypeStruct((_E, 32), _f32),
            jax.ShapeDtypeStruct((_E, 16), _f32)]
    out_specs = [pl.BlockSpec((_BE, o.shape[1]), lambda i: (i, 0))
                 for o in outs]
    return pl.pallas_call(
        _edge_body, grid=(nblk,), in_specs=[espec] + wspecs,
        out_specs=out_specs, out_shape=outs)(
            e_h, wc0, bc0, wc1, bc1, wc2, bc2, wo, bo)


_BN = 2000


def _node_body(x_ref, a0_ref, a1_ref, w1, b1, w2, b2, out_ref, *, use_gelu,
               single):
    if single:
        t = x_ref[...] + a0_ref[...]
    else:
        t = x_ref[...] + a0_ref[...] + a1_ref[...]
    h = jnp.dot(t, w1[...], preferred_element_type=_f32) + b1[...]
    h = jnp.maximum(h, 0.0)
    h = jnp.dot(h, w2[...], preferred_element_type=_f32) + b2[...]
    if use_gelu:
        h = 0.5 * h * (1.0 + lax.erf(h * 0.7071067811865476))
    out_ref[...] = h


def _node_update(x, a0, a1, W1, b1, W2, b2, use_gelu):
    din = x.shape[1]
    dhid = W1.shape[1]
    dout = W2.shape[1]
    nblk = _N // _BN
    bspec = lambda d: pl.BlockSpec((_BN, d), lambda i: (i, 0))
    single = a1 is None
    args = [x, a0] + ([] if single else [a1])
    in_specs = [bspec(din), bspec(din)] + ([] if single else [bspec(din)])
    if single:
        def body(x_ref, a0_ref, w1, b1, w2, b2, out_ref):
            _node_body(x_ref, a0_ref, None, w1, b1, w2, b2, out_ref,
                       use_gelu=use_gelu, single=True)
    else:
        body = functools.partial(_node_body, use_gelu=use_gelu, single=False)
    return pl.pallas_call(
        body,
        grid=(nblk,),
        in_specs=in_specs + [_full((din, dhid)), _full((1, dhid)),
                             _full((dhid, dout)), _full((1, dout))],
        out_specs=bspec(dout),
        out_shape=jax.ShapeDtypeStruct((_N, dout), _f32))(
            *args, W1, b1.reshape(1, -1), W2, b2.reshape(1, -1))


_sc_l0 = _sc_layer0()
_sc_l12 = _sc_layer12()


def kernel(inc_node_edge, x_h, e_h, edge_in_W, edge_in_b, gnn_params,
           edge_out_W, edge_out_b):
    src = inc_node_edge[:, 0]
    dst = inc_node_edge[:, 1]
    src16 = src.reshape(16, 2, _NCH0 // 2, _C)
    dst16 = dst.reshape(16, 2, _NCH0 // 2, _C)
    src32 = src.reshape(_NW, _NCH, _C)
    dstq = (dst // 4).reshape(_NW, _NCH, _C)
    dstm = jnp.broadcast_to(((dst % 4) * 32)[:, None], (_E, 16))
    dstm = dstm.reshape(_NW, _NCH, _C // 8, 128)

    (wc0, bc0, wc1, bc1, wc2, bc2, wo, bo) = _fold_weights(
        edge_in_W, edge_in_b, gnn_params, edge_out_W, edge_out_b)

    el0, el1, el2, eo = _edge_features(e_h, wc0, bc0, wc1, bc1, wc2, bc2,
                                       wo, bo)

    p = gnn_params[0]
    agg = _sc_l0(x_h, el0.reshape(16, _NCH0, _C, 128), src16, dst16)
    agg = agg[:, :5000].reshape(_N, 128)
    x = _node_update(x_h, agg, None, p["W1"], p["b1"], p["W2"], p["b2"],
                     use_gelu=True)
    xs = [x]

    for l in (1, 2):
        p = gnn_params[l]
        el = (el1 if l == 1 else el2).reshape(_NW, _NCH, _C // 4, 128)
        xrep = jnp.tile(x, (1, 4))
        ag = _sc_l12(xrep, el, src32, dstq, dstm)
        a0 = ag[0].reshape(4 * _PK_ROWS, 32)[:_N]
        a1 = ag[1].reshape(4 * _PK_ROWS, 32)[:_N]
        x = _node_update(x, a0, a1, p["W1"], p["b1"], p["W2"], p["b2"],
                         use_gelu=(l < 2))
        xs.append(x)

    return (x, eo, tuple(xs))
